# baseline (speedup 1.0000x reference)
import jax
import jax.numpy as jnp
from jax import lax
from jax.experimental import pallas as pl
from jax.experimental.pallas import tpu as pltpu
from jax.experimental.pallas import tpu_sc as plsc

B = 16384
EMB = 128
NC = 2
NS = 16
NW = NC * NS
BPW = B // NW
CHUNK = 32
NCHUNK = BPW // CHUNK


def _sc_body(users_hbm, items_hbm, negs_hbm, umem_hbm, imem_hbm, vw_hbm,
             pos_hbm, neg_hbm,
             uidx_v, iidx_v, nidx_v,
             u_b0, i_b0, n_b0, u_b1, i_b1, n_b1,
             v_v, pos_v, neg_v, sem0, sem1):
    wid = lax.axis_index("s") * NC + lax.axis_index("c")
    base = wid * BPW
    pltpu.sync_copy(vw_hbm.at[0], v_v)
    pltpu.sync_copy(users_hbm.at[pl.ds(base, BPW)], uidx_v)
    pltpu.sync_copy(items_hbm.at[pl.ds(base, BPW)], iidx_v)
    pltpu.sync_copy(negs_hbm.at[pl.ds(base, BPW)], nidx_v)

    bufs = [(u_b0, i_b0, n_b0, sem0), (u_b1, i_b1, n_b1, sem1)]

    def start(c, parity):
        ub, ib, nb, sem = bufs[parity]
        sl = pl.ds(c * CHUNK, CHUNK)
        pltpu.async_copy(umem_hbm.at[uidx_v.at[sl]], ub, sem)
        pltpu.async_copy(imem_hbm.at[iidx_v.at[sl]], ib, sem)
        pltpu.async_copy(imem_hbm.at[nidx_v.at[sl]], nb, sem)

    def drain(parity):
        ub, ib, nb, sem = bufs[parity]
        dummy = umem_hbm.at[pl.ds(0, CHUNK)]
        pltpu.make_async_copy(dummy, ub, sem).wait()
        pltpu.make_async_copy(dummy, ib, sem).wait()
        pltpu.make_async_copy(dummy, nb, sem).wait()

    vj = [v_v[pl.ds(16 * j, 16)] for j in range(EMB // 16)]
    last_lane = lax.iota(jnp.int32, 16) == 15

    def compute_chunk(c, parity):
        urows_v, irows_v, nrows_v, _ = bufs[parity]
        outbase = c * CHUNK

        @plsc.parallel_loop(0, CHUNK, 1, unroll=2,
                            carry=jnp.full((16,), outbase, jnp.int32))
        def row_body(r, ridx):
            sls = [pl.ds(16 * j, 16) for j in range(EMB // 16)]
            us = [urows_v[r, sl] for sl in sls]
            is_ = [irows_v[r, sl] for sl in sls]
            ns = [nrows_v[r, sl] for sl in sls]
            ts = [us[j] * vj[j] for j in range(EMB // 16)]
            ap0 = ap1 = an0 = an1 = None
            for j in range(EMB // 16):
                p = ts[j] * is_[j]
                n = ts[j] * ns[j]
                if j % 2 == 0:
                    ap0 = p if ap0 is None else ap0 + p
                    an0 = n if an0 is None else an0 + n
                else:
                    ap1 = p if ap1 is None else ap1 + p
                    an1 = n if an1 is None else an1 + n
            sp = jnp.cumsum(ap0 + ap1)
            sn = jnp.cumsum(an0 + an1)
            plsc.store_scatter(pos_v, [ridx], jnp.maximum(sp, 0.0),
                               mask=last_lane)
            plsc.store_scatter(neg_v, [ridx], jnp.maximum(sn, 0.0),
                               mask=last_lane)
            return ridx + 1

    start(0, 0)
    start(1, 1)

    def pair_body(p, carry):
        c0 = 2 * p
        for parity in range(2):
            c = c0 + parity
            drain(parity)
            compute_chunk(c, parity)

            @pl.when(c + 2 < NCHUNK)
            def _():
                start(c + 2, parity)

        return carry

    lax.fori_loop(0, NCHUNK // 2, pair_body, 0)

    pltpu.sync_copy(pos_v, pos_hbm.at[pl.ds(base, BPW)])
    pltpu.sync_copy(neg_v, neg_hbm.at[pl.ds(base, BPW)])


@jax.jit
def _run(users, items, negs, umem, imem, vw):
    f = pl.kernel(
        _sc_body,
        out_type=(
            jax.ShapeDtypeStruct((B,), jnp.float32),
            jax.ShapeDtypeStruct((B,), jnp.float32),
        ),
        mesh=plsc.VectorSubcoreMesh(core_axis_name="c", subcore_axis_name="s"),
        compiler_params=pltpu.CompilerParams(use_tc_tiling_on_sc=False,
                                             needs_layout_passes=False),
        scratch_types=[
            pltpu.VMEM((BPW,), jnp.int32),
            pltpu.VMEM((BPW,), jnp.int32),
            pltpu.VMEM((BPW,), jnp.int32),
            pltpu.VMEM((CHUNK, EMB), jnp.float32),
            pltpu.VMEM((CHUNK, EMB), jnp.float32),
            pltpu.VMEM((CHUNK, EMB), jnp.float32),
            pltpu.VMEM((CHUNK, EMB), jnp.float32),
            pltpu.VMEM((CHUNK, EMB), jnp.float32),
            pltpu.VMEM((CHUNK, EMB), jnp.float32),
            pltpu.VMEM((EMB,), jnp.float32),
            pltpu.VMEM((BPW,), jnp.float32),
            pltpu.VMEM((BPW,), jnp.float32),
            pltpu.SemaphoreType.DMA,
            pltpu.SemaphoreType.DMA,
        ],
    )
    return f(users, items, negs, umem, imem, vw)


def kernel(input_users, input_items, input_items_negative, user_memory,
           item_memory, v_w):
    pos, neg = _run(input_users.astype(jnp.int32),
                    input_items.astype(jnp.int32),
                    input_items_negative.astype(jnp.int32),
                    user_memory, item_memory, v_w)
    return pos.reshape(B, 1), neg.reshape(B, 1)


# speedup vs baseline: 1.0391x; 1.0391x over previous
import jax
import jax.numpy as jnp
from jax import lax
from jax.experimental import pallas as pl
from jax.experimental.pallas import tpu as pltpu
from jax.experimental.pallas import tpu_sc as plsc

B = 16384
EMB = 128
NC = 2
NS = 16
NW = NC * NS
BPW = B // NW
CHUNK = 64
NCHUNK = BPW // CHUNK


def _sc_body(users_hbm, items_hbm, negs_hbm, umem_hbm, imem_hbm, vw_hbm,
             pos_hbm, neg_hbm,
             uidx_v, iidx_v, nidx_v,
             u_b0, i_b0, n_b0, u_b1, i_b1, n_b1,
             v_v, pos_v, neg_v, sem0, sem1):
    wid = lax.axis_index("s") * NC + lax.axis_index("c")
    base = wid * BPW
    pltpu.sync_copy(vw_hbm.at[0], v_v)
    pltpu.sync_copy(users_hbm.at[pl.ds(base, BPW)], uidx_v)
    pltpu.sync_copy(items_hbm.at[pl.ds(base, BPW)], iidx_v)
    pltpu.sync_copy(negs_hbm.at[pl.ds(base, BPW)], nidx_v)

    bufs = [(u_b0, i_b0, n_b0, sem0), (u_b1, i_b1, n_b1, sem1)]

    def start(c, parity):
        ub, ib, nb, sem = bufs[parity]
        sl = pl.ds(c * CHUNK, CHUNK)
        pltpu.async_copy(umem_hbm.at[uidx_v.at[sl]], ub, sem)
        pltpu.async_copy(imem_hbm.at[iidx_v.at[sl]], ib, sem)
        pltpu.async_copy(imem_hbm.at[nidx_v.at[sl]], nb, sem)

    def drain(parity):
        ub, ib, nb, sem = bufs[parity]
        dummy = umem_hbm.at[pl.ds(0, CHUNK)]
        pltpu.make_async_copy(dummy, ub, sem).wait()
        pltpu.make_async_copy(dummy, ib, sem).wait()
        pltpu.make_async_copy(dummy, nb, sem).wait()

    vj = [v_v[pl.ds(16 * j, 16)] for j in range(EMB // 16)]
    last_lane = lax.iota(jnp.int32, 16) == 15

    def compute_chunk(c, parity):
        urows_v, irows_v, nrows_v, _ = bufs[parity]
        outbase = c * CHUNK

        @plsc.parallel_loop(0, CHUNK, 1, unroll=4,
                            carry=jnp.full((16,), outbase, jnp.int32))
        def row_body(r, ridx):
            sls = [pl.ds(16 * j, 16) for j in range(EMB // 16)]
            us = [urows_v[r, sl] for sl in sls]
            is_ = [irows_v[r, sl] for sl in sls]
            ns = [nrows_v[r, sl] for sl in sls]
            ts = [us[j] * vj[j] for j in range(EMB // 16)]
            ap0 = ap1 = an0 = an1 = None
            for j in range(EMB // 16):
                p = ts[j] * is_[j]
                n = ts[j] * ns[j]
                if j % 2 == 0:
                    ap0 = p if ap0 is None else ap0 + p
                    an0 = n if an0 is None else an0 + n
                else:
                    ap1 = p if ap1 is None else ap1 + p
                    an1 = n if an1 is None else an1 + n
            sp = jnp.cumsum(ap0 + ap1)
            sn = jnp.cumsum(an0 + an1)
            plsc.store_scatter(pos_v, [ridx], jnp.maximum(sp, 0.0),
                               mask=last_lane)
            plsc.store_scatter(neg_v, [ridx], jnp.maximum(sn, 0.0),
                               mask=last_lane)
            return ridx + 1

    start(0, 0)
    start(1, 1)

    def pair_body(p, carry):
        c0 = 2 * p
        for parity in range(2):
            c = c0 + parity
            drain(parity)
            compute_chunk(c, parity)

            @pl.when(c + 2 < NCHUNK)
            def _():
                start(c + 2, parity)

        return carry

    lax.fori_loop(0, NCHUNK // 2, pair_body, 0)

    pltpu.sync_copy(pos_v, pos_hbm.at[pl.ds(base, BPW)])
    pltpu.sync_copy(neg_v, neg_hbm.at[pl.ds(base, BPW)])


@jax.jit
def _run(users, items, negs, umem, imem, vw):
    f = pl.kernel(
        _sc_body,
        out_type=(
            jax.ShapeDtypeStruct((B,), jnp.float32),
            jax.ShapeDtypeStruct((B,), jnp.float32),
        ),
        mesh=plsc.VectorSubcoreMesh(core_axis_name="c", subcore_axis_name="s"),
        compiler_params=pltpu.CompilerParams(use_tc_tiling_on_sc=False,
                                             needs_layout_passes=False),
        scratch_types=[
            pltpu.VMEM((BPW,), jnp.int32),
            pltpu.VMEM((BPW,), jnp.int32),
            pltpu.VMEM((BPW,), jnp.int32),
            pltpu.VMEM((CHUNK, EMB), jnp.float32),
            pltpu.VMEM((CHUNK, EMB), jnp.float32),
            pltpu.VMEM((CHUNK, EMB), jnp.float32),
            pltpu.VMEM((CHUNK, EMB), jnp.float32),
            pltpu.VMEM((CHUNK, EMB), jnp.float32),
            pltpu.VMEM((CHUNK, EMB), jnp.float32),
            pltpu.VMEM((EMB,), jnp.float32),
            pltpu.VMEM((BPW,), jnp.float32),
            pltpu.VMEM((BPW,), jnp.float32),
            pltpu.SemaphoreType.DMA,
            pltpu.SemaphoreType.DMA,
        ],
    )
    return f(users, items, negs, umem, imem, vw)


def kernel(input_users, input_items, input_items_negative, user_memory,
           item_memory, v_w):
    pos, neg = _run(input_users.astype(jnp.int32),
                    input_items.astype(jnp.int32),
                    input_items_negative.astype(jnp.int32),
                    user_memory, item_memory, v_w)
    return pos.reshape(B, 1), neg.reshape(B, 1)


# speedup vs baseline: 1.0523x; 1.0127x over previous
import jax
import jax.numpy as jnp
from jax import lax
from jax.experimental import pallas as pl
from jax.experimental.pallas import tpu as pltpu
from jax.experimental.pallas import tpu_sc as plsc

B = 16384
EMB = 128
NC = 2
NS = 16
NW = NC * NS
BPW = B // NW
CHUNK = 64
NCHUNK = BPW // CHUNK


def _sc_body(users_hbm, items_hbm, negs_hbm, umem_hbm, imem_hbm, vw_hbm,
             pos_hbm, neg_hbm,
             uidx_v, iidx_v, nidx_v,
             u_b0, i_b0, n_b0, u_b1, i_b1, n_b1,
             v_v, pos_v, neg_v, sem0, sem1):
    wid = lax.axis_index("s") * NC + lax.axis_index("c")
    base = wid * BPW
    pltpu.sync_copy(vw_hbm.at[0], v_v)
    pltpu.sync_copy(users_hbm.at[pl.ds(base, BPW)], uidx_v)
    pltpu.sync_copy(items_hbm.at[pl.ds(base, BPW)], iidx_v)
    pltpu.sync_copy(negs_hbm.at[pl.ds(base, BPW)], nidx_v)

    bufs = [(u_b0, i_b0, n_b0, sem0), (u_b1, i_b1, n_b1, sem1)]

    def start(c, parity):
        ub, ib, nb, sem = bufs[parity]
        sl = pl.ds(c * CHUNK, CHUNK)
        pltpu.async_copy(umem_hbm.at[uidx_v.at[sl]], ub, sem)
        pltpu.async_copy(imem_hbm.at[iidx_v.at[sl]], ib, sem)
        pltpu.async_copy(imem_hbm.at[nidx_v.at[sl]], nb, sem)

    def drain(parity):
        ub, ib, nb, sem = bufs[parity]
        dummy = umem_hbm.at[pl.ds(0, CHUNK)]
        pltpu.make_async_copy(dummy, ub, sem).wait()
        pltpu.make_async_copy(dummy, ib, sem).wait()
        pltpu.make_async_copy(dummy, nb, sem).wait()

    vj = [v_v[pl.ds(16 * j, 16)] for j in range(EMB // 16)]
    last_lane = lax.iota(jnp.int32, 16) == 15

    def compute_chunk(c, parity):
        urows_v, irows_v, nrows_v, _ = bufs[parity]
        outbase = c * CHUNK

        @plsc.parallel_loop(0, CHUNK, 1, unroll=2,
                            carry=jnp.full((16,), outbase, jnp.int32))
        def row_body(r, ridx):
            sls = [pl.ds(16 * j, 16) for j in range(EMB // 16)]
            us = [urows_v[r, sl] for sl in sls]
            is_ = [irows_v[r, sl] for sl in sls]
            ns = [nrows_v[r, sl] for sl in sls]
            ts = [us[j] * vj[j] for j in range(EMB // 16)]
            ap0 = ap1 = an0 = an1 = None
            for j in range(EMB // 16):
                p = ts[j] * is_[j]
                n = ts[j] * ns[j]
                if j % 2 == 0:
                    ap0 = p if ap0 is None else ap0 + p
                    an0 = n if an0 is None else an0 + n
                else:
                    ap1 = p if ap1 is None else ap1 + p
                    an1 = n if an1 is None else an1 + n
            sp = jnp.cumsum(ap0 + ap1)
            sn = jnp.cumsum(an0 + an1)
            plsc.store_scatter(pos_v, [ridx], jnp.maximum(sp, 0.0),
                               mask=last_lane)
            plsc.store_scatter(neg_v, [ridx], jnp.maximum(sn, 0.0),
                               mask=last_lane)
            return ridx + 1

    start(0, 0)
    start(1, 1)

    def pair_body(p, carry):
        c0 = 2 * p
        for parity in range(2):
            c = c0 + parity
            drain(parity)
            compute_chunk(c, parity)

            @pl.when(c + 2 < NCHUNK)
            def _():
                start(c + 2, parity)

        return carry

    lax.fori_loop(0, NCHUNK // 2, pair_body, 0)

    pltpu.sync_copy(pos_v, pos_hbm.at[pl.ds(base, BPW)])
    pltpu.sync_copy(neg_v, neg_hbm.at[pl.ds(base, BPW)])


@jax.jit
def _run(users, items, negs, umem, imem, vw):
    f = pl.kernel(
        _sc_body,
        out_type=(
            jax.ShapeDtypeStruct((B,), jnp.float32),
            jax.ShapeDtypeStruct((B,), jnp.float32),
        ),
        mesh=plsc.VectorSubcoreMesh(core_axis_name="c", subcore_axis_name="s"),
        compiler_params=pltpu.CompilerParams(use_tc_tiling_on_sc=False,
                                             needs_layout_passes=False),
        scratch_types=[
            pltpu.VMEM((BPW,), jnp.int32),
            pltpu.VMEM((BPW,), jnp.int32),
            pltpu.VMEM((BPW,), jnp.int32),
            pltpu.VMEM((CHUNK, EMB), jnp.float32),
            pltpu.VMEM((CHUNK, EMB), jnp.float32),
            pltpu.VMEM((CHUNK, EMB), jnp.float32),
            pltpu.VMEM((CHUNK, EMB), jnp.float32),
            pltpu.VMEM((CHUNK, EMB), jnp.float32),
            pltpu.VMEM((CHUNK, EMB), jnp.float32),
            pltpu.VMEM((EMB,), jnp.float32),
            pltpu.VMEM((BPW,), jnp.float32),
            pltpu.VMEM((BPW,), jnp.float32),
            pltpu.SemaphoreType.DMA,
            pltpu.SemaphoreType.DMA,
        ],
    )
    return f(users, items, negs, umem, imem, vw)


def kernel(input_users, input_items, input_items_negative, user_memory,
           item_memory, v_w):
    pos, neg = _run(input_users.astype(jnp.int32),
                    input_items.astype(jnp.int32),
                    input_items_negative.astype(jnp.int32),
                    user_memory, item_memory, v_w)
    return pos.reshape(B, 1), neg.reshape(B, 1)


# speedup vs baseline: 1.0762x; 1.0227x over previous
import jax
import jax.numpy as jnp
from jax import lax
from jax.experimental import pallas as pl
from jax.experimental.pallas import tpu as pltpu
from jax.experimental.pallas import tpu_sc as plsc

B = 16384
EMB = 128
NC = 2
NS = 16
NW = NC * NS
BPW = B // NW
CHUNK = 64
NCHUNK = BPW // CHUNK


def _sc_body(users_hbm, items_hbm, negs_hbm, umem_hbm, imem_hbm, vw_hbm,
             pos_hbm, neg_hbm,
             uidx_v, iidx_v, nidx_v,
             u_b0, i_b0, n_b0, u_b1, i_b1, n_b1,
             v_v, pos_v, neg_v, sem0, sem1):
    wid = lax.axis_index("s") * NC + lax.axis_index("c")
    base = wid * BPW
    pltpu.sync_copy(users_hbm.at[pl.ds(base, CHUNK)], uidx_v.at[pl.ds(0, CHUNK)])
    pltpu.sync_copy(items_hbm.at[pl.ds(base, CHUNK)], iidx_v.at[pl.ds(0, CHUNK)])
    pltpu.sync_copy(negs_hbm.at[pl.ds(base, CHUNK)], nidx_v.at[pl.ds(0, CHUNK)])

    bufs = [(u_b0, i_b0, n_b0, sem0), (u_b1, i_b1, n_b1, sem1)]

    def start(c, parity):
        ub, ib, nb, sem = bufs[parity]
        sl = pl.ds(c * CHUNK, CHUNK)
        pltpu.async_copy(umem_hbm.at[uidx_v.at[sl]], ub, sem)
        pltpu.async_copy(imem_hbm.at[iidx_v.at[sl]], ib, sem)
        pltpu.async_copy(imem_hbm.at[nidx_v.at[sl]], nb, sem)

    def drain(parity):
        ub, ib, nb, sem = bufs[parity]
        dummy = umem_hbm.at[pl.ds(0, CHUNK)]
        pltpu.make_async_copy(dummy, ub, sem).wait()
        pltpu.make_async_copy(dummy, ib, sem).wait()
        pltpu.make_async_copy(dummy, nb, sem).wait()

    vj = [v_v[pl.ds(16 * j, 16)] for j in range(EMB // 16)]
    last_lane = lax.iota(jnp.int32, 16) == 15

    def compute_chunk(c, parity):
        urows_v, irows_v, nrows_v, _ = bufs[parity]
        outbase = c * CHUNK

        @plsc.parallel_loop(0, CHUNK, 1, unroll=2,
                            carry=jnp.full((16,), outbase, jnp.int32))
        def row_body(r, ridx):
            sls = [pl.ds(16 * j, 16) for j in range(EMB // 16)]
            us = [urows_v[r, sl] for sl in sls]
            is_ = [irows_v[r, sl] for sl in sls]
            ns = [nrows_v[r, sl] for sl in sls]
            ts = [us[j] * vj[j] for j in range(EMB // 16)]
            ap0 = ap1 = an0 = an1 = None
            for j in range(EMB // 16):
                p = ts[j] * is_[j]
                n = ts[j] * ns[j]
                if j % 2 == 0:
                    ap0 = p if ap0 is None else ap0 + p
                    an0 = n if an0 is None else an0 + n
                else:
                    ap1 = p if ap1 is None else ap1 + p
                    an1 = n if an1 is None else an1 + n
            sp = jnp.cumsum(ap0 + ap1)
            sn = jnp.cumsum(an0 + an1)
            plsc.store_scatter(pos_v, [ridx], jnp.maximum(sp, 0.0),
                               mask=last_lane)
            plsc.store_scatter(neg_v, [ridx], jnp.maximum(sn, 0.0),
                               mask=last_lane)
            return ridx + 1

    start(0, 0)
    rest = BPW - CHUNK
    pltpu.sync_copy(users_hbm.at[pl.ds(base + CHUNK, rest)],
                    uidx_v.at[pl.ds(CHUNK, rest)])
    pltpu.sync_copy(items_hbm.at[pl.ds(base + CHUNK, rest)],
                    iidx_v.at[pl.ds(CHUNK, rest)])
    pltpu.sync_copy(negs_hbm.at[pl.ds(base + CHUNK, rest)],
                    nidx_v.at[pl.ds(CHUNK, rest)])
    pltpu.sync_copy(vw_hbm.at[0], v_v)
    start(1, 1)

    def pair_body(p, carry):
        c0 = 2 * p
        for parity in range(2):
            c = c0 + parity
            drain(parity)
            compute_chunk(c, parity)

            @pl.when(c + 2 < NCHUNK)
            def _():
                start(c + 2, parity)

        return carry

    lax.fori_loop(0, NCHUNK // 2, pair_body, 0)

    pltpu.sync_copy(pos_v, pos_hbm.at[pl.ds(base, BPW)])
    pltpu.sync_copy(neg_v, neg_hbm.at[pl.ds(base, BPW)])


@jax.jit
def _run(users, items, negs, umem, imem, vw):
    f = pl.kernel(
        _sc_body,
        out_type=(
            jax.ShapeDtypeStruct((B,), jnp.float32),
            jax.ShapeDtypeStruct((B,), jnp.float32),
        ),
        mesh=plsc.VectorSubcoreMesh(core_axis_name="c", subcore_axis_name="s"),
        compiler_params=pltpu.CompilerParams(use_tc_tiling_on_sc=False,
                                             needs_layout_passes=False),
        scratch_types=[
            pltpu.VMEM((BPW,), jnp.int32),
            pltpu.VMEM((BPW,), jnp.int32),
            pltpu.VMEM((BPW,), jnp.int32),
            pltpu.VMEM((CHUNK, EMB), jnp.float32),
            pltpu.VMEM((CHUNK, EMB), jnp.float32),
            pltpu.VMEM((CHUNK, EMB), jnp.float32),
            pltpu.VMEM((CHUNK, EMB), jnp.float32),
            pltpu.VMEM((CHUNK, EMB), jnp.float32),
            pltpu.VMEM((CHUNK, EMB), jnp.float32),
            pltpu.VMEM((EMB,), jnp.float32),
            pltpu.VMEM((BPW,), jnp.float32),
            pltpu.VMEM((BPW,), jnp.float32),
            pltpu.SemaphoreType.DMA,
            pltpu.SemaphoreType.DMA,
        ],
    )
    return f(users, items, negs, umem, imem, vw)


def kernel(input_users, input_items, input_items_negative, user_memory,
           item_memory, v_w):
    pos, neg = _run(input_users.astype(jnp.int32),
                    input_items.astype(jnp.int32),
                    input_items_negative.astype(jnp.int32),
                    user_memory, item_memory, v_w)
    return pos.reshape(B, 1), neg.reshape(B, 1)
